# Initial kernel scaffold; baseline (speedup 1.0000x reference)
import jax
import jax.numpy as jnp
from jax.experimental import pallas as pl


def kernel(user_text, user_feats, graph_node_features, graph_edge_index, merged_tree_feature, merged_tree_edge_index, indices, h0_graph, h0_tree, params):
    raise NotImplementedError("write your pallas kernel here")


# speedup vs baseline: 2.0397x; 2.0397x over previous
import functools

import jax
import jax.numpy as jnp
from jax import lax
from jax.experimental import pallas as pl
from jax.experimental.pallas import tpu as pltpu

HID = 100
SEQ = 8
NBLK = 1000


def _gru_body(x_ref, h0_ref, w0i_ref, w0h_ref, b0i_ref, b0h_ref,
              w1i_ref, w1h_ref, b1i_ref, b1h_ref, out_ref):
    h0 = h0_ref[0]
    h1 = h0_ref[1]
    b0i = b0i_ref[...]
    b0h = b0h_ref[...]
    b1i = b1i_ref[...]
    b1h = b1h_ref[...]
    w0i = w0i_ref[...]
    w0h = w0h_ref[...]
    w1i = w1i_ref[...]
    w1h = w1h_ref[...]

    def cell(x, h, wi, wh, bi, bh):
        gi = jnp.dot(x, wi, preferred_element_type=jnp.float32) + bi
        gh = jnp.dot(h, wh, preferred_element_type=jnp.float32) + bh
        ir, iz, inn = gi[:, :HID], gi[:, HID:2 * HID], gi[:, 2 * HID:]
        hr, hz, hn = gh[:, :HID], gh[:, HID:2 * HID], gh[:, 2 * HID:]
        r = jax.nn.sigmoid(ir + hr)
        z = jax.nn.sigmoid(iz + hz)
        n = jnp.tanh(inn + r * hn)
        return (1.0 - z) * n + z * h

    for t in range(SEQ):
        x0 = cell(x_ref[:, t, :], h0, w0i, w0h, b0i, b0h)
        h0 = x0
        h1 = cell(x0, h1, w1i, w1h, b1i, b1h)
    out_ref[...] = h1


def _gru2(te, gru_p, h0):
    n = te.shape[0]
    grid = n // NBLK
    wspec = pl.BlockSpec((HID, 3 * HID), lambda i: (0, 0))
    bspec = pl.BlockSpec((1, 3 * HID), lambda i: (0, 0))
    args = (
        te,
        h0,
        gru_p['w_ih_0'].T, gru_p['w_hh_0'].T,
        gru_p['b_ih_0'][None], gru_p['b_hh_0'][None],
        gru_p['w_ih_1'].T, gru_p['w_hh_1'].T,
        gru_p['b_ih_1'][None], gru_p['b_hh_1'][None],
    )
    return pl.pallas_call(
        _gru_body,
        grid=(grid,),
        in_specs=[
            pl.BlockSpec((NBLK, SEQ, HID), lambda i: (i, 0, 0)),
            pl.BlockSpec((2, NBLK, HID), lambda i: (0, i, 0)),
            wspec, wspec, bspec, bspec, wspec, wspec, bspec, bspec,
        ],
        out_specs=pl.BlockSpec((NBLK, HID), lambda i: (i, 0)),
        out_shape=jax.ShapeDtypeStruct((n, HID), jnp.float32),
    )(*args)


def _proj_body(x_ref, b_ref, w_ref, as_ref, ad_ref, xw_ref, s_ref, d_ref,
               *, relu_in):
    x = x_ref[...]
    if relu_in:
        x = jnp.maximum(x + b_ref[...], 0.0)
    xw = jnp.dot(x, w_ref[...], preferred_element_type=jnp.float32)
    xw_ref[...] = xw
    s_ref[...] = jnp.dot(xw, as_ref[...], preferred_element_type=jnp.float32)
    d_ref[...] = jnp.dot(xw, ad_ref[...], preferred_element_type=jnp.float32)


def _proj(x, bias_in, w, a_src, a_dst, relu_in):
    n, fin = x.shape
    fout = w.shape[1]
    h, c = a_src.shape
    blockdiag = jnp.zeros((h, c, h), jnp.float32)
    a_s_m = blockdiag.at[jnp.arange(h), :, jnp.arange(h)].set(a_src).reshape(fout, h)
    a_d_m = blockdiag.at[jnp.arange(h), :, jnp.arange(h)].set(a_dst).reshape(fout, h)
    if bias_in is None:
        bias_in = jnp.zeros((1, fin), jnp.float32)
    else:
        bias_in = bias_in[None]
    grid = n // NBLK
    xw, a_s, a_d = pl.pallas_call(
        functools.partial(_proj_body, relu_in=relu_in),
        grid=(grid,),
        in_specs=[
            pl.BlockSpec((NBLK, fin), lambda i: (i, 0)),
            pl.BlockSpec((1, fin), lambda i: (0, 0)),
            pl.BlockSpec((fin, fout), lambda i: (0, 0)),
            pl.BlockSpec((fout, h), lambda i: (0, 0)),
            pl.BlockSpec((fout, h), lambda i: (0, 0)),
        ],
        out_specs=[
            pl.BlockSpec((NBLK, fout), lambda i: (i, 0)),
            pl.BlockSpec((NBLK, h), lambda i: (i, 0)),
            pl.BlockSpec((NBLK, h), lambda i: (i, 0)),
        ],
        out_shape=[
            jax.ShapeDtypeStruct((n, fout), jnp.float32),
            jax.ShapeDtypeStruct((n, h), jnp.float32),
            jax.ShapeDtypeStruct((n, h), jnp.float32),
        ],
    )(x, bias_in, w, a_s_m, a_d_m)
    return xw, a_s, a_d


def _mlp_body(x_ref, w1_ref, b1_ref, w2_ref, b2_ref, o_ref):
    h = jnp.dot(x_ref[...], w1_ref[...], preferred_element_type=jnp.float32)
    h = jnp.maximum(h + b1_ref[...], 0.0)
    o_ref[...] = jnp.dot(h, w2_ref[...], preferred_element_type=jnp.float32) + b2_ref[...]


def _user_mlp(user_feats, p):
    n = user_feats.shape[0]
    return pl.pallas_call(
        _mlp_body,
        out_shape=jax.ShapeDtypeStruct((n, HID), jnp.float32),
    )(user_feats, p['ue_fc1_w'].T, p['ue_fc1_b'][None],
      p['ue_fc2_w'].T, p['ue_fc2_b'][None])


def _segmean_body(child_ref, b_ref, idx_ref, fcw_ref, fcb_ref, o_ref,
                  sums_ref, cnt_ref, *, nblocks, bseg):
    i = pl.program_id(0)

    @pl.when(i == 0)
    def _init():
        sums_ref[...] = jnp.zeros_like(sums_ref)
        cnt_ref[...] = jnp.zeros_like(cnt_ref)

    c = jnp.maximum(child_ref[...] + b_ref[...], 0.0)
    seg = idx_ref[0, 0].astype(jnp.int32)
    onehot = (seg[:, None] == lax.broadcasted_iota(jnp.int32, (1, bseg), 1)
              ).astype(jnp.float32)
    sums_ref[...] += jnp.dot(onehot.T, c, preferred_element_type=jnp.float32)
    cnt_ref[...] += jnp.sum(onehot, axis=0, keepdims=True)

    @pl.when(i == nblocks - 1)
    def _fin():
        mean = sums_ref[...] / jnp.maximum(cnt_ref[...], 1.0).T
        o_ref[...] = jnp.dot(mean, fcw_ref[...],
                             preferred_element_type=jnp.float32) + fcb_ref[...]


def _segmean_fc(child_raw, bias, indices, fc_w, fc_b, bseg):
    n, f = child_raw.shape
    grid = n // NBLK
    return pl.pallas_call(
        functools.partial(_segmean_body, nblocks=grid, bseg=bseg),
        grid=(grid,),
        in_specs=[
            pl.BlockSpec((NBLK, f), lambda i: (i, 0)),
            pl.BlockSpec((1, f), lambda i: (0, 0)),
            pl.BlockSpec((1, 1, NBLK), lambda i: (i, 0, 0)),
            pl.BlockSpec((f, fc_w.shape[0]), lambda i: (0, 0)),
            pl.BlockSpec((1, fc_w.shape[0]), lambda i: (0, 0)),
        ],
        out_specs=pl.BlockSpec((bseg, fc_w.shape[0]), lambda i: (0, 0)),
        out_shape=jax.ShapeDtypeStruct((bseg, fc_w.shape[0]), jnp.float32),
        scratch_shapes=[
            pltpu.VMEM((bseg, f), jnp.float32),
            pltpu.VMEM((1, bseg), jnp.float32),
        ],
    )(child_raw, bias[None], indices.astype(jnp.int32).reshape(grid, 1, NBLK),
      fc_w.T, fc_b[None])


def _gat_edges(xw, a_s, a_d, src, dst, n, heads, c):
    e = jax.nn.leaky_relu(a_s[src] + a_d[dst], negative_slope=0.2)
    m = jax.ops.segment_max(e, dst, num_segments=n)
    e = jnp.exp(e - m[dst])
    s = jax.ops.segment_sum(e, dst, num_segments=n)
    alpha = e / (s[dst] + 1e-16)
    msg = xw[src].reshape(-1, heads, c) * alpha[:, :, None]
    return jax.ops.segment_sum(msg.reshape(-1, heads * c), dst, num_segments=n)


def kernel(user_text, user_feats, graph_node_features, graph_edge_index,
           merged_tree_feature, merged_tree_edge_index, indices,
           h0_graph, h0_tree, params):
    p = params
    b = h0_graph.shape[1] and 128
    n_graph = 10000
    n_tree = h0_tree.shape[1]

    user_embedding = _user_mlp(user_feats, p)

    te = jnp.take(p['tweet_table'], graph_node_features, axis=0)
    hn = _gru2(te, p['g_gru'], h0_graph)
    x_input = jnp.concatenate([hn[:b], user_embedding, hn[b:]], axis=0)

    loop_g = jnp.arange(n_graph, dtype=graph_edge_index.dtype)
    src_g = jnp.concatenate([graph_edge_index[0], loop_g])
    dst_g = jnp.concatenate([graph_edge_index[1], loop_g])

    xw1, as1, ad1 = _proj(x_input, None, p['g_conv1']['W'],
                          p['g_conv1']['a_src'], p['g_conv1']['a_dst'], False)
    sum1 = _gat_edges(xw1, as1, ad1, src_g, dst_g, n_graph, 8, 64)

    xw2, as2, ad2 = _proj(sum1, p['g_conv1']['bias'], p['g_conv2']['W'],
                          p['g_conv2']['a_src'], p['g_conv2']['a_dst'], True)
    sum2 = _gat_edges(xw2, as2, ad2, src_g, dst_g, n_graph, 1, 100)
    user_root = jnp.maximum(sum2 + p['g_conv2']['bias'], 0.0)

    tt = jnp.take(p['tweet_table'], merged_tree_feature, axis=0)
    hn_t = _gru2(tt, p['t_gru'], h0_tree)
    x_in = jnp.concatenate([user_root[:b], hn_t[b:]], axis=0)

    loop_t = jnp.arange(n_tree, dtype=merged_tree_edge_index.dtype)
    src_t = jnp.concatenate([merged_tree_edge_index[0], loop_t])
    dst_t = jnp.concatenate([merged_tree_edge_index[1], loop_t])

    xw3, as3, ad3 = _proj(x_in, None, p['t_conv1']['W'],
                          p['t_conv1']['a_src'], p['t_conv1']['a_dst'], False)
    sum3 = _gat_edges(xw3, as3, ad3, src_t, dst_t, n_tree, 8, 100)

    xw4, as4, ad4 = _proj(sum3, p['t_conv1']['bias'], p['t_conv2']['W'],
                          p['t_conv2']['a_src'], p['t_conv2']['a_dst'], True)
    sum4 = _gat_edges(xw4, as4, ad4, src_t, dst_t, n_tree, 1, 100)

    return _segmean_fc(sum4, p['t_conv2']['bias'], indices,
                       p['fc_w'], p['fc_b'], b)


# speedup vs baseline: 3.3002x; 1.6180x over previous
import functools

import jax
import jax.numpy as jnp
from jax import lax
from jax.experimental import pallas as pl
from jax.experimental.pallas import tpu as pltpu
from jax.experimental.pallas import tpu_sc as plsc

HID = 100
SEQ = 8
NBLK = 1000
PBLK = 640
N1 = 10240
EPAD = 172032
EBLK = 128
NTILE = 16
HALF = N1 // 2
SAC = HALF
ACCR = HALF + 64
NZC = ACCR // 64
NOC = HALF // NTILE // 64


def _gru_body(x_ref, h0_ref, w0i_ref, w0h_ref, b0i_ref, b0h_ref,
              w1i_ref, w1h_ref, b1i_ref, b1h_ref, out_ref):
    h0 = h0_ref[0]
    h1 = h0_ref[1]

    def cell(x, h, wi, wh, bi, bh):
        gi = jnp.dot(x, wi, preferred_element_type=jnp.float32) + bi
        gh = jnp.dot(h, wh, preferred_element_type=jnp.float32) + bh
        ir, iz, inn = gi[:, :HID], gi[:, HID:2 * HID], gi[:, 2 * HID:]
        hr, hz, hn = gh[:, :HID], gh[:, HID:2 * HID], gh[:, 2 * HID:]
        r = jax.nn.sigmoid(ir + hr)
        z = jax.nn.sigmoid(iz + hz)
        n = jnp.tanh(inn + r * hn)
        return (1.0 - z) * n + z * h

    for t in range(SEQ):
        x0 = cell(x_ref[:, t, :], h0, w0i_ref[...], w0h_ref[...],
                  b0i_ref[...], b0h_ref[...])
        h0 = x0
        h1 = cell(x0, h1, w1i_ref[...], w1h_ref[...],
                  b1i_ref[...], b1h_ref[...])
    out_ref[...] = h1


def _gru2(te, gru_p, h0):
    n = te.shape[0]
    grid = n // NBLK
    wspec = pl.BlockSpec((HID, 3 * HID), lambda i: (0, 0))
    bspec = pl.BlockSpec((1, 3 * HID), lambda i: (0, 0))
    return pl.pallas_call(
        _gru_body,
        grid=(grid,),
        in_specs=[
            pl.BlockSpec((NBLK, SEQ, HID), lambda i: (i, 0, 0)),
            pl.BlockSpec((2, NBLK, HID), lambda i: (0, i, 0)),
            wspec, wspec, bspec, bspec, wspec, wspec, bspec, bspec,
        ],
        out_specs=pl.BlockSpec((NBLK, HID), lambda i: (i, 0)),
        out_shape=jax.ShapeDtypeStruct((n, HID), jnp.float32),
    )(te, h0,
      gru_p['w_ih_0'].T, gru_p['w_hh_0'].T,
      gru_p['b_ih_0'][None], gru_p['b_hh_0'][None],
      gru_p['w_ih_1'].T, gru_p['w_hh_1'].T,
      gru_p['b_ih_1'][None], gru_p['b_hh_1'][None])


def _att_mats(a_src, a_dst, cdim):
    h = a_src.shape[0]
    cp = 64 if cdim == 64 else 128
    bd = jnp.zeros((h, cp, 128), jnp.float32)
    ar = jnp.arange(h)
    asm = bd.at[ar, :cdim, ar].set(a_src).reshape(h * cp, 128)
    adm = bd.at[ar, :cdim, ar].set(a_dst).reshape(h * cp, 128)
    return asm, adm


def _pad_w(w, h, cdim):
    if cdim == 64:
        return w
    fin = w.shape[0]
    return jnp.pad(w.reshape(fin, h, cdim),
                   ((0, 0), (0, 0), (0, 128 - cdim))).reshape(fin, h * 128)


def _proj_body(x_ref, w_ref, as_ref, ad_ref, xw_ref, s_ref, d_ref):
    xw = jnp.dot(x_ref[...], w_ref[...], preferred_element_type=jnp.float32)
    xw_ref[...] = xw
    s_ref[...] = jnp.dot(xw, as_ref[...], preferred_element_type=jnp.float32)
    d_ref[...] = jnp.dot(xw, ad_ref[...], preferred_element_type=jnp.float32)


def _proj(x, w, a_src, a_dst, cdim):
    n, fin = x.shape
    h = a_src.shape[0]
    wp = _pad_w(w, h, cdim)
    fp = wp.shape[1]
    asm, adm = _att_mats(a_src, a_dst, cdim)
    grid = n // PBLK
    return pl.pallas_call(
        _proj_body,
        grid=(grid,),
        in_specs=[
            pl.BlockSpec((PBLK, fin), lambda i: (i, 0)),
            pl.BlockSpec((fin, fp), lambda i: (0, 0)),
            pl.BlockSpec((fp, 128), lambda i: (0, 0)),
            pl.BlockSpec((fp, 128), lambda i: (0, 0)),
        ],
        out_specs=[
            pl.BlockSpec((PBLK, fp), lambda i: (i, 0)),
            pl.BlockSpec((PBLK, 128), lambda i: (i, 0)),
            pl.BlockSpec((PBLK, 128), lambda i: (i, 0)),
        ],
        out_shape=[
            jax.ShapeDtypeStruct((n, fp), jnp.float32),
            jax.ShapeDtypeStruct((n, 128), jnp.float32),
            jax.ShapeDtypeStruct((n, 128), jnp.float32),
        ],
    )(x, wp, asm, adm)


def _proj2_body(*refs, nchunks, fcv):
    parts = refs[:nchunks]
    b_ref, w_ref, as_ref, ad_ref, xw_ref, s_ref, d_ref = refs[nchunks:]
    acc = None
    for c in range(nchunks):
        p = parts[c]
        x = jnp.maximum(p[:, :fcv] + b_ref[:, c * fcv:(c + 1) * fcv], 0.0)
        contrib = jnp.dot(x, w_ref[c * fcv:(c + 1) * fcv, :],
                          preferred_element_type=jnp.float32)
        acc = contrib if acc is None else acc + contrib
    xw_ref[...] = acc
    s_ref[...] = jnp.dot(acc, as_ref[...], preferred_element_type=jnp.float32)
    d_ref[...] = jnp.dot(acc, ad_ref[...], preferred_element_type=jnp.float32)


def _proj2(parts, fcv, bias, w, a_src, a_dst, cdim):
    nchunks = len(parts)
    fin = w.shape[0]
    h = a_src.shape[0]
    wp = _pad_w(w, h, cdim)
    fp = wp.shape[1]
    asm, adm = _att_mats(a_src, a_dst, cdim)
    grid = N1 // PBLK
    part_specs = [pl.BlockSpec((PBLK, 128), lambda i: (i, 0))
                  for _ in range(nchunks)]
    return pl.pallas_call(
        functools.partial(_proj2_body, nchunks=nchunks, fcv=fcv),
        grid=(grid,),
        in_specs=part_specs + [
            pl.BlockSpec((1, fin), lambda i: (0, 0)),
            pl.BlockSpec((fin, fp), lambda i: (0, 0)),
            pl.BlockSpec((fp, 128), lambda i: (0, 0)),
            pl.BlockSpec((fp, 128), lambda i: (0, 0)),
        ],
        out_specs=[
            pl.BlockSpec((PBLK, fp), lambda i: (i, 0)),
            pl.BlockSpec((PBLK, 128), lambda i: (i, 0)),
            pl.BlockSpec((PBLK, 128), lambda i: (i, 0)),
        ],
        out_shape=[
            jax.ShapeDtypeStruct((N1, fp), jnp.float32),
            jax.ShapeDtypeStruct((N1, 128), jnp.float32),
            jax.ShapeDtypeStruct((N1, 128), jnp.float32),
        ],
    )(*parts, bias[None], wp, asm, adm)


def _user_root_body(p_ref, b_ref, o_ref):
    o_ref[...] = jnp.maximum(p_ref[:, :HID] + b_ref[...], 0.0)


def _user_root(part, bias, nrows):
    return pl.pallas_call(
        _user_root_body,
        grid=(1,),
        in_specs=[
            pl.BlockSpec((nrows, 128), lambda i: (0, 0)),
            pl.BlockSpec((1, HID), lambda i: (0, 0)),
        ],
        out_specs=pl.BlockSpec((nrows, HID), lambda i: (0, 0)),
        out_shape=jax.ShapeDtypeStruct((nrows, HID), jnp.float32),
    )(part, bias[None])


def _mlp_body(x_ref, w1_ref, b1_ref, w2_ref, b2_ref, o_ref):
    h = jnp.dot(x_ref[...], w1_ref[...], preferred_element_type=jnp.float32)
    h = jnp.maximum(h + b1_ref[...], 0.0)
    o_ref[...] = jnp.dot(h, w2_ref[...],
                         preferred_element_type=jnp.float32) + b2_ref[...]


def _user_mlp(user_feats, p):
    n = user_feats.shape[0]
    return pl.pallas_call(
        _mlp_body,
        out_shape=jax.ShapeDtypeStruct((n, HID), jnp.float32),
    )(user_feats, p['ue_fc1_w'].T, p['ue_fc1_b'][None],
      p['ue_fc2_w'].T, p['ue_fc2_b'][None])


def _segmean_body(child_ref, b_ref, idx_ref, fcw_ref, fcb_ref, o_ref,
                  sums_ref, cnt_ref, *, nblocks, bseg):
    i = pl.program_id(0)

    @pl.when(i == 0)
    def _init():
        sums_ref[...] = jnp.zeros_like(sums_ref)
        cnt_ref[...] = jnp.zeros_like(cnt_ref)

    c = jnp.maximum(child_ref[:, :HID] + b_ref[...], 0.0)
    seg = idx_ref[0, 0].astype(jnp.int32)
    onehot = (seg[:, None] == lax.broadcasted_iota(jnp.int32, (1, bseg), 1)
              ).astype(jnp.float32)
    sums_ref[...] += jnp.dot(onehot.T, c, preferred_element_type=jnp.float32)
    cnt_ref[...] += jnp.sum(onehot, axis=0, keepdims=True)

    @pl.when(i == nblocks - 1)
    def _fin():
        mean = sums_ref[...] / jnp.maximum(cnt_ref[...], 1.0).T
        o_ref[...] = jnp.dot(mean, fcw_ref[...],
                             preferred_element_type=jnp.float32) + fcb_ref[...]


def _segmean_fc(child_part, bias, indices, fc_w, fc_b, bseg):
    n = indices.shape[0]
    grid = n // NBLK
    return pl.pallas_call(
        functools.partial(_segmean_body, nblocks=grid, bseg=bseg),
        grid=(grid,),
        in_specs=[
            pl.BlockSpec((NBLK, 128), lambda i: (i, 0)),
            pl.BlockSpec((1, HID), lambda i: (0, 0)),
            pl.BlockSpec((1, 1, NBLK), lambda i: (i, 0, 0)),
            pl.BlockSpec((HID, fc_w.shape[0]), lambda i: (0, 0)),
            pl.BlockSpec((1, fc_w.shape[0]), lambda i: (0, 0)),
        ],
        out_specs=pl.BlockSpec((bseg, fc_w.shape[0]), lambda i: (0, 0)),
        out_shape=jax.ShapeDtypeStruct((bseg, fc_w.shape[0]), jnp.float32),
        scratch_shapes=[
            pltpu.VMEM((bseg, HID), jnp.float32),
            pltpu.VMEM((1, bseg), jnp.float32),
        ],
    )(child_part, bias[None], indices.astype(jnp.int32).reshape(grid, 1, NBLK),
      fc_w.T, fc_b[None])


def _sc_mesh():
    return plsc.VectorSubcoreMesh(core_axis_name="c", subcore_axis_name="s")


def _iota16():
    return lax.iota(jnp.int32, 16)


def _sc_phase1(a_s, a_d, src, dst, heads):
    h = heads
    bpt = EPAD // EBLK // NTILE
    ge = EBLK * h // 16
    zeros = jnp.zeros((64, 128), jnp.float32)

    @functools.partial(
        pl.kernel, mesh=_sc_mesh(),
        out_type=(jax.ShapeDtypeStruct((EPAD, h), jnp.float32),
                  jax.ShapeDtypeStruct((N1, 128), jnp.float32)),
        scratch_types=[
            pltpu.VMEM((EBLK,), jnp.int32),
            pltpu.VMEM((EBLK,), jnp.int32),
            pltpu.VMEM((EBLK,), jnp.int32),
            pltpu.VMEM((EBLK, 128), jnp.float32),
            pltpu.VMEM((EBLK, 128), jnp.float32),
            pltpu.VMEM((EBLK, h), jnp.float32),
            pltpu.VMEM((EBLK, 128), jnp.float32),
            pltpu.VMEM((64, 128), jnp.float32),
            pltpu.VMEM_SHARED((ACCR, 128), jnp.float32),
            pltpu.SemaphoreType.DMA,
            pltpu.SemaphoreType.DMA,
        ],
        compiler_params=pltpu.CompilerParams(needs_layout_passes=False),
    )
    def k(as_hbm, ad_hbm, src_hbm, dst_hbm, z_hbm, e_hbm, s_hbm,
          src_v, dst_v, dstm_v, as_v, ad_v, e_v, e_v2, zv, s_sp, sem1, sem2):
        core = lax.axis_index("c")
        tid = lax.axis_index("s")
        rbase = core * HALF

        pltpu.sync_copy(z_hbm, zv)
        pltpu.sync_copy(z_hbm, e_v2.at[pl.ds(0, 64)])
        pltpu.sync_copy(z_hbm, e_v2.at[pl.ds(64, 64)])
        for j in range(NZC):
            @pl.when(tid == j % NTILE)
            def _z(j=j):
                pltpu.sync_copy(zv, s_sp.at[pl.ds(j * 64, 64)])

        plsc.subcore_barrier()

        def blk_body(blk, carry):
            eb = (tid * bpt + blk) * EBLK
            pltpu.sync_copy(src_hbm.at[pl.ds(eb, EBLK)], src_v)
            pltpu.sync_copy(dst_hbm.at[pl.ds(eb, EBLK)], dst_v)
            cp1 = pltpu.async_copy(as_hbm.at[src_v], as_v, sem1)
            cp2 = pltpu.async_copy(ad_hbm.at[dst_v], ad_v, sem2)
            for g in range(EBLK // 16):
                dv = dst_v[pl.ds(g * 16, 16)] - rbase
                ok = (dv >= 0) & (dv < HALF)
                dstm_v[pl.ds(g * 16, 16)] = jnp.where(ok, dv, SAC)
            cp1.wait()
            cp2.wait()

            def g_body(g, c2):
                f16 = g * 16 + _iota16()
                r16 = f16 // h
                c16 = f16 % h
                x = (plsc.load_gather(as_v, [r16, c16])
                     + plsc.load_gather(ad_v, [r16, c16]))
                x = jnp.where(x > 0.0, x, 0.2 * x)
                ev = jnp.exp(x)
                plsc.store_scatter(e_v, [r16, c16], ev)
                plsc.store_scatter(e_v2, [r16, c16], ev)
                return c2

            lax.fori_loop(0, ge, g_body, 0)

            @pl.when(core == 0)
            def _we():
                pltpu.sync_copy(e_v, e_hbm.at[pl.ds(eb, EBLK)])

            pltpu.sync_copy(e_v2, s_sp.at[dstm_v], add=True)
            return carry

        lax.fori_loop(0, bpt, blk_body, 0)
        plsc.subcore_barrier()
        for rb in range(NOC):
            base = (tid * NOC + rb) * 64
            pltpu.sync_copy(s_sp.at[pl.ds(base, 64)], as_v.at[pl.ds(0, 64)])
            pltpu.sync_copy(as_v.at[pl.ds(0, 64)],
                            s_hbm.at[pl.ds(rbase + base, 64)])

    return k(a_s, a_d, src, dst, zeros)


def _sc_phase2(src, dst, e_all, s_all, xw_chunk, heads, cdim, head0):
    h = heads
    bpt = EPAD // EBLK // NTILE
    ge = EBLK * h // 16
    gm = EBLK * 128 // 16
    zeros = jnp.zeros((64, 128), jnp.float32)

    @functools.partial(
        pl.kernel, mesh=_sc_mesh(),
        out_type=jax.ShapeDtypeStruct((N1, 128), jnp.float32),
        scratch_types=[
            pltpu.VMEM((EBLK,), jnp.int32),
            pltpu.VMEM((EBLK,), jnp.int32),
            pltpu.VMEM((EBLK,), jnp.int32),
            pltpu.VMEM((EBLK, h), jnp.float32),
            pltpu.VMEM((EBLK, 128), jnp.float32),
            pltpu.VMEM((EBLK, h), jnp.float32),
            pltpu.VMEM((EBLK, 128), jnp.float32),
            pltpu.VMEM((64, 128), jnp.float32),
            pltpu.VMEM_SHARED((ACCR, 128), jnp.float32),
            pltpu.SemaphoreType.DMA,
            pltpu.SemaphoreType.DMA,
        ],
        compiler_params=pltpu.CompilerParams(needs_layout_passes=False),
    )
    def k(src_hbm, dst_hbm, e_hbm, s_hbm, xw_hbm, z_hbm, out_hbm,
          src_v, dst_v, dstm_v, e_v, s_v, a_v, msg_v, zv, outbuf, sem1, sem2):
        core = lax.axis_index("c")
        tid = lax.axis_index("s")
        rbase = core * HALF
        pltpu.sync_copy(z_hbm, zv)
        for j in range(NZC):
            @pl.when(tid == j % NTILE)
            def _z(j=j):
                pltpu.sync_copy(zv, outbuf.at[pl.ds(j * 64, 64)])
        plsc.subcore_barrier()

        def blk_body(blk, carry):
            eb = (tid * bpt + blk) * EBLK
            pltpu.sync_copy(src_hbm.at[pl.ds(eb, EBLK)], src_v)
            pltpu.sync_copy(dst_hbm.at[pl.ds(eb, EBLK)], dst_v)
            cp1 = pltpu.async_copy(xw_hbm.at[src_v], msg_v, sem1)
            cp2 = pltpu.async_copy(s_hbm.at[dst_v], s_v, sem2)
            pltpu.sync_copy(e_hbm.at[pl.ds(eb, EBLK)], e_v)
            for g in range(EBLK // 16):
                dv = dst_v[pl.ds(g * 16, 16)] - rbase
                ok = (dv >= 0) & (dv < HALF)
                dstm_v[pl.ds(g * 16, 16)] = jnp.where(ok, dv, SAC)
            cp1.wait()
            cp2.wait()

            def a_body(g, c2):
                f16 = g * 16 + _iota16()
                r16 = f16 // h
                c16 = f16 % h
                ev = plsc.load_gather(e_v, [r16, c16])
                sv = plsc.load_gather(s_v, [r16, c16])
                al = ev / (sv + 1e-16)
                plsc.store_scatter(a_v, [r16, c16], al)
                return c2

            lax.fori_loop(0, ge, a_body, 0)

            def m_body(g, c2):
                f16 = g * 16 + _iota16()
                r16 = f16 // 128
                c16 = f16 % 128
                if cdim == 64:
                    h16 = head0 + c16 // 64
                else:
                    h16 = jnp.full((16,), head0, jnp.int32)
                m = plsc.load_gather(msg_v, [r16, c16])
                al = plsc.load_gather(a_v, [r16, h16])
                plsc.store_scatter(msg_v, [r16, c16], m * al)
                return c2

            lax.fori_loop(0, gm, m_body, 0)
            pltpu.sync_copy(msg_v, outbuf.at[dstm_v], add=True)
            return carry

        lax.fori_loop(0, bpt, blk_body, 0)
        plsc.subcore_barrier()
        for rb in range(NOC):
            base = (tid * NOC + rb) * 64
            pltpu.sync_copy(outbuf.at[pl.ds(base, 64)], msg_v.at[pl.ds(0, 64)])
            pltpu.sync_copy(msg_v.at[pl.ds(0, 64)],
                            out_hbm.at[pl.ds(rbase + base, 64)])

    return k(src, dst, e_all, s_all, xw_chunk, zeros)


def _sc_gat(xw_pad, a_s, a_d, src, dst, heads, cdim):
    e_all, s_all = _sc_phase1(a_s, a_d, src, dst, heads)
    nchunks = xw_pad.shape[1] // 128
    hpc = 2 if cdim == 64 else 1
    parts = []
    for c in range(nchunks):
        parts.append(
            _sc_phase2(src, dst, e_all, s_all,
                       xw_pad[:, c * 128:(c + 1) * 128],
                       heads, cdim, c * hpc))
    return parts


def _pad_edges(edge_index, n):
    e = edge_index.shape[1]
    loop = jnp.arange(n, dtype=jnp.int32)
    fill = jnp.full((EPAD - e - n,), n, jnp.int32)
    src = jnp.concatenate([edge_index[0].astype(jnp.int32), loop, fill])
    dst = jnp.concatenate([edge_index[1].astype(jnp.int32), loop, fill])
    return src, dst


def _pad_nodes(x):
    return jnp.pad(x, ((0, N1 - x.shape[0]), (0, 0)))


def kernel(user_text, user_feats, graph_node_features, graph_edge_index,
           merged_tree_feature, merged_tree_edge_index, indices,
           h0_graph, h0_tree, params):
    p = params
    b = 128
    n_graph = 10000
    n_tree = h0_tree.shape[1]

    user_embedding = _user_mlp(user_feats, p)

    te = jnp.take(p['tweet_table'], graph_node_features, axis=0)
    hn = _gru2(te, p['g_gru'], h0_graph)
    x_input = jnp.concatenate([hn[:b], user_embedding, hn[b:]], axis=0)

    src_g, dst_g = _pad_edges(graph_edge_index, n_graph)

    xw1, as1, ad1 = _proj(_pad_nodes(x_input), p['g_conv1']['W'],
                          p['g_conv1']['a_src'], p['g_conv1']['a_dst'], 64)
    parts1 = _sc_gat(xw1, as1, ad1, src_g, dst_g, 8, 64)

    xw2, as2, ad2 = _proj2(parts1, 128, p['g_conv1']['bias'],
                           p['g_conv2']['W'], p['g_conv2']['a_src'],
                           p['g_conv2']['a_dst'], 100)
    parts2 = _sc_gat(xw2, as2, ad2, src_g, dst_g, 1, 100)
    user_root = _user_root(parts2[0], p['g_conv2']['bias'], b)

    tt = jnp.take(p['tweet_table'], merged_tree_feature, axis=0)
    hn_t = _gru2(tt, p['t_gru'], h0_tree)
    x_in = jnp.concatenate([user_root, hn_t[b:]], axis=0)

    src_t, dst_t = _pad_edges(merged_tree_edge_index, n_tree)

    xw3, as3, ad3 = _proj(_pad_nodes(x_in), p['t_conv1']['W'],
                          p['t_conv1']['a_src'], p['t_conv1']['a_dst'], 100)
    parts3 = _sc_gat(xw3, as3, ad3, src_t, dst_t, 8, 100)

    xw4, as4, ad4 = _proj2(parts3, 100, p['t_conv1']['bias'],
                           p['t_conv2']['W'], p['t_conv2']['a_src'],
                           p['t_conv2']['a_dst'], 100)
    parts4 = _sc_gat(xw4, as4, ad4, src_t, dst_t, 1, 100)

    return _segmean_fc(parts4[0], p['t_conv2']['bias'], indices,
                       p['fc_w'], p['fc_b'], b)
